# Initial kernel scaffold; baseline (speedup 1.0000x reference)
import jax
import jax.numpy as jnp
from jax.experimental import pallas as pl


def kernel(obj_vecs, pred_vecs, edges, W1, b1, W2, b2, W3, b3, W4, b4):
    raise NotImplementedError("write your pallas kernel here")


# speedup vs baseline: 3145.7259x; 3145.7259x over previous
import functools

import jax
import jax.numpy as jnp
from jax import lax
from jax.experimental import pallas as pl
from jax.experimental.pallas import tpu as pltpu
from jax.experimental.pallas import tpu_sc as plsc

O = 10000
T = 160000
D = 384
H = 384

NC = 2
NS = 16
NW = NC * NS

C = 128
GPW = T // NW
GCH = GPW // C
GTL = GPW - GCH * C
SPT = T // NS
SCH = SPT // C
STL = SPT - SCH * C
HALF = O // 2
ACC_ROWS = 5120
DUMMY = HALF
PW = 128
NP = H // PW


def _proj_table(obj, W1r):
    nO = 10
    bO = O // nO

    def body(w_ref, x_ref, o_ref):
        o_ref[...] = jnp.dot(x_ref[...], w_ref[0],
                             preferred_element_type=jnp.float32)

    return pl.pallas_call(
        body,
        grid=(2, nO),
        in_specs=[
            pl.BlockSpec((1, D, H), lambda g, j: (2 * g, 0, 0)),
            pl.BlockSpec((bO, D), lambda g, j: (j, 0)),
        ],
        out_specs=pl.BlockSpec((bO, H), lambda g, j: (g * nO + j, 0)),
        out_shape=jax.ShapeDtypeStruct((2 * O, H), jnp.float32),
    )(W1r, obj)


def _sc_gather(table, idx_main, idx_tail):
    mesh = plsc.VectorSubcoreMesh(core_axis_name="c", subcore_axis_name="s")

    @functools.partial(
        pl.kernel,
        mesh=mesh,
        out_type=(
            jax.ShapeDtypeStruct((T, H), jnp.float32),
            jax.ShapeDtypeStruct((T, H), jnp.float32),
        ),
        scratch_types=[
            pltpu.VMEM((GCH, C), jnp.int32),
            pltpu.VMEM((GCH, C), jnp.int32),
            pltpu.VMEM((2, GTL), jnp.int32),
            pltpu.VMEM((C, H), jnp.float32),
            pltpu.VMEM((C, H), jnp.float32),
            pltpu.SemaphoreType.DMA,
            pltpu.SemaphoreType.DMA,
        ],
    )
    def k(table_hbm, idxm_hbm, idxt_hbm, outA_hbm, outC_hbm,
          idx_va, idx_vc, idx_vt, rows_a, rows_c, sem_a, sem_c):
        wid = lax.axis_index("s") * NC + lax.axis_index("c")
        base = wid * GPW
        pltpu.sync_copy(idxm_hbm.at[0, wid], idx_va)
        pltpu.sync_copy(idxm_hbm.at[1, wid], idx_vc)
        pltpu.sync_copy(idxt_hbm.at[0, wid], idx_vt.at[0])
        pltpu.sync_copy(idxt_hbm.at[1, wid], idx_vt.at[1])

        def body(j, carry):
            cp_a = pltpu.async_copy(table_hbm.at[idx_va.at[j]], rows_a, sem_a)
            cp_c = pltpu.async_copy(table_hbm.at[idx_vc.at[j]], rows_c, sem_c)
            cp_a.wait()
            pltpu.sync_copy(rows_a, outA_hbm.at[pl.ds(base + j * C, C)])
            cp_c.wait()
            pltpu.sync_copy(rows_c, outC_hbm.at[pl.ds(base + j * C, C)])
            return carry

        lax.fori_loop(0, GCH, body, 0)

        cp_a = pltpu.async_copy(table_hbm.at[idx_vt.at[0]],
                                rows_a.at[pl.ds(0, GTL)], sem_a)
        cp_c = pltpu.async_copy(table_hbm.at[idx_vt.at[1]],
                                rows_c.at[pl.ds(0, GTL)], sem_c)
        cp_a.wait()
        pltpu.sync_copy(rows_a.at[pl.ds(0, GTL)],
                        outA_hbm.at[pl.ds(base + GCH * C, GTL)])
        cp_c.wait()
        pltpu.sync_copy(rows_c.at[pl.ds(0, GTL)],
                        outC_hbm.at[pl.ds(base + GCH * C, GTL)])

    return k(table, idx_main, idx_tail)


def _edge_mlp(gA, gC, pred, W1b, b1r, W2, b2r):
    tile = 640
    n = T // tile

    def body(ga, gc, pr, w1, b1_, w2, b2_, os_, op_, oo_):
        h = ga[...] + gc[...] + b1_[...]
        h = h + jnp.dot(pr[...], w1[...], preferred_element_type=jnp.float32)
        h = jnp.maximum(h, 0.0)
        nt = jnp.dot(h, w2[...], preferred_element_type=jnp.float32) + b2_[...]
        nt = jnp.maximum(nt, 0.0)
        os_[...] = nt[:, :H]
        op_[...] = nt[:, H:H + D]
        oo_[...] = nt[:, H + D:]

    edge_spec = pl.BlockSpec((tile, D), lambda j: (j, 0))
    return pl.pallas_call(
        body,
        grid=(n,),
        in_specs=[
            edge_spec, edge_spec, edge_spec,
            pl.BlockSpec((D, H), lambda j: (0, 0)),
            pl.BlockSpec((1, H), lambda j: (0, 0)),
            pl.BlockSpec((H, 2 * H + D), lambda j: (0, 0)),
            pl.BlockSpec((1, 2 * H + D), lambda j: (0, 0)),
        ],
        out_specs=[edge_spec, edge_spec, edge_spec],
        out_shape=[
            jax.ShapeDtypeStruct((T, H), jnp.float32),
            jax.ShapeDtypeStruct((T, D), jnp.float32),
            jax.ShapeDtypeStruct((T, H), jnp.float32),
        ],
        compiler_params=pltpu.CompilerParams(
            dimension_semantics=("arbitrary",)),
    )(gA, gC, pred, W1b, b1r, W2, b2r)


def _sc_scatter_add(new_s, new_o, idx_main, idx_tail):
    mesh = plsc.VectorSubcoreMesh(core_axis_name="c", subcore_axis_name="s")

    @functools.partial(
        pl.kernel,
        mesh=mesh,
        out_type=jax.ShapeDtypeStruct((O, H), jnp.float32),
        scratch_types=[
            pltpu.VMEM((2, SCH, C), jnp.int32),
            pltpu.VMEM((2, STL), jnp.int32),
            pltpu.VMEM((C, PW), jnp.float32),
            pltpu.VMEM_SHARED((ACC_ROWS, PW), jnp.float32),
        ],
    )
    def k(s_hbm, o_hbm, idxm_hbm, idxt_hbm, out_hbm,
          idx_v, idx_vt, rows_v, acc):
        c = lax.axis_index("c")
        s = lax.axis_index("s")

        pltpu.sync_copy(idxm_hbm.at[0, c, s], idx_v.at[0])
        pltpu.sync_copy(idxm_hbm.at[1, c, s], idx_v.at[1])
        pltpu.sync_copy(idxt_hbm.at[0, c, s], idx_vt.at[0])
        pltpu.sync_copy(idxt_hbm.at[1, c, s], idx_vt.at[1])

        zpt = ACC_ROWS // NS
        cpt = HALF // NS

        for p in range(NP):
            def zrow(r, carry):
                for kk in range(PW // 16):
                    rows_v[r, pl.ds(kk * 16, 16)] = jnp.zeros(
                        (16,), jnp.float32)
                return carry

            lax.fori_loop(0, C, zrow, 0)
            for z0, zr in ((0, C), (C, C), (2 * C, zpt - 2 * C)):
                pltpu.sync_copy(rows_v.at[pl.ds(0, zr)],
                                acc.at[pl.ds(s * zpt + z0, zr)])
            plsc.subcore_barrier()

            col = pl.ds(p * PW, PW)

            def body_s(j, carry):
                pltpu.sync_copy(s_hbm.at[pl.ds(s * SPT + j * C, C), col],
                                rows_v)
                pltpu.sync_copy(rows_v, acc.at[idx_v.at[0, j]], add=True)
                return carry

            lax.fori_loop(0, SCH, body_s, 0)

            def body_o(j, carry):
                pltpu.sync_copy(o_hbm.at[pl.ds(s * SPT + j * C, C), col],
                                rows_v)
                pltpu.sync_copy(rows_v, acc.at[idx_v.at[1, j]], add=True)
                return carry

            lax.fori_loop(0, SCH, body_o, 0)

            pltpu.sync_copy(s_hbm.at[pl.ds(s * SPT + SCH * C, STL), col],
                            rows_v.at[pl.ds(0, STL)])
            pltpu.sync_copy(rows_v.at[pl.ds(0, STL)],
                            acc.at[idx_vt.at[0]], add=True)
            pltpu.sync_copy(o_hbm.at[pl.ds(s * SPT + SCH * C, STL), col],
                            rows_v.at[pl.ds(0, STL)])
            pltpu.sync_copy(rows_v.at[pl.ds(0, STL)],
                            acc.at[idx_vt.at[1]], add=True)
            plsc.subcore_barrier()

            pltpu.sync_copy(acc.at[pl.ds(s * cpt, cpt)],
                            out_hbm.at[pl.ds(c * HALF + s * cpt, cpt), col])

            @pl.when(s == NS - 1)
            def _():
                pltpu.sync_copy(
                    acc.at[pl.ds(NS * cpt, HALF - NS * cpt)],
                    out_hbm.at[pl.ds(c * HALF + NS * cpt,
                                     HALF - NS * cpt), col])

            plsc.subcore_barrier()

    return k(new_s, new_o, idx_main, idx_tail)


def _sumsq(pooled):
    n = 25
    b = O // n

    def body(x_ref, o_ref, acc_ref):
        @pl.when(pl.program_id(0) == 0)
        def _():
            acc_ref[0] = 0.0

        x = x_ref[...]
        acc_ref[0] += jnp.sum(x * x)

        @pl.when(pl.program_id(0) == n - 1)
        def _():
            o_ref[...] = jnp.broadcast_to(acc_ref[0], (1, 1))

    return pl.pallas_call(
        body,
        grid=(n,),
        in_specs=[pl.BlockSpec((b, H), lambda j: (j, 0))],
        out_specs=pl.BlockSpec((1, 1), lambda j: (0, 0)),
        out_shape=jax.ShapeDtypeStruct((1, 1), jnp.float32),
        scratch_shapes=[pltpu.SMEM((1,), jnp.float32)],
        compiler_params=pltpu.CompilerParams(
            dimension_semantics=("arbitrary",)),
    )(pooled)


def _gconv2(pooled, ss, W3, b3r, W4, b4r):
    n = 25
    b = O // n

    def body(ss_ref, x_ref, w3, b3_, w4, b4_, o_ref):
        inv = lax.rsqrt(ss_ref[0, 0])
        h = jnp.dot(x_ref[...] * inv, w3[...],
                    preferred_element_type=jnp.float32) + b3_[...]
        h = jnp.maximum(h, 0.0)
        o = jnp.dot(h, w4[...], preferred_element_type=jnp.float32) + b4_[...]
        o_ref[...] = jnp.maximum(o, 0.0)

    return pl.pallas_call(
        body,
        grid=(n,),
        in_specs=[
            pl.BlockSpec((1, 1), lambda j: (0, 0)),
            pl.BlockSpec((b, H), lambda j: (j, 0)),
            pl.BlockSpec((H, H), lambda j: (0, 0)),
            pl.BlockSpec((1, H), lambda j: (0, 0)),
            pl.BlockSpec((H, D), lambda j: (0, 0)),
            pl.BlockSpec((1, D), lambda j: (0, 0)),
        ],
        out_specs=pl.BlockSpec((b, D), lambda j: (j, 0)),
        out_shape=jax.ShapeDtypeStruct((O, D), jnp.float32),
    )(ss, pooled, W3, b3r, W4, b4r)


def kernel(obj_vecs, pred_vecs, edges, W1, b1, W2, b2, W3, b3, W4, b4):
    obj = obj_vecs[0]
    pred = pred_vecs[0]
    s_idx = edges[0, :, 0].astype(jnp.int32)
    o_idx = edges[0, :, 1].astype(jnp.int32)

    W1r = W1.reshape(3, D, H)

    table = _proj_table(obj, W1r)

    idx_g = jnp.stack([s_idx, o_idx + O]).reshape(2, NW, GPW)
    idx_gm = idx_g[:, :, :GCH * C].reshape(2, NW, GCH, C)
    idx_gt = idx_g[:, :, GCH * C:]
    gA, gC = _sc_gather(table, idx_gm, idx_gt)

    new_s, new_pred, new_o = _edge_mlp(
        gA, gC, pred, W1r[1], b1.reshape(1, H), W2, b2.reshape(1, 2 * H + D))

    def eff(ix, c):
        lo = c * HALF
        return jnp.where((ix >= lo) & (ix < lo + HALF), ix - lo, DUMMY)

    idx_sc = jnp.stack([
        jnp.stack([eff(s_idx, 0), eff(s_idx, 1)]),
        jnp.stack([eff(o_idx, 0), eff(o_idx, 1)]),
    ]).reshape(2, NC, NS, SPT)
    idx_sm = idx_sc[..., :SCH * C].reshape(2, NC, NS, SCH, C)
    idx_st = idx_sc[..., SCH * C:]
    pooled = _sc_scatter_add(new_s, new_o, idx_sm, idx_st)

    ss = _sumsq(pooled)
    new_obj = _gconv2(pooled, ss, W3, b3.reshape(1, H), W4, b4.reshape(1, D))

    return new_obj[None], new_pred[None]


# speedup vs baseline: 4705.6109x; 1.4959x over previous
import functools

import jax
import jax.numpy as jnp
from jax import lax
from jax.experimental import pallas as pl
from jax.experimental.pallas import tpu as pltpu
from jax.experimental.pallas import tpu_sc as plsc

O = 10000
T = 160000
D = 384
H = 384

NC = 2
NS = 16
NW = NC * NS

C = 128
GPW = T // NW
GCH = GPW // C
GTL = GPW - GCH * C
TPC = T // NC
SPT = TPC // NS
SCH = SPT // C
STL = SPT - SCH * C
PW = 128
NP = H // PW
CPT = 624


def _proj_table(obj, W1r):
    nO = 10
    bO = O // nO

    def body(w_ref, x_ref, o_ref):
        o_ref[...] = jnp.dot(x_ref[...], w_ref[0],
                             preferred_element_type=jnp.float32)

    return pl.pallas_call(
        body,
        grid=(2, nO),
        in_specs=[
            pl.BlockSpec((1, D, H), lambda g, j: (2 * g, 0, 0)),
            pl.BlockSpec((bO, D), lambda g, j: (j, 0)),
        ],
        out_specs=pl.BlockSpec((bO, H), lambda g, j: (g * nO + j, 0)),
        out_shape=jax.ShapeDtypeStruct((2 * O, H), jnp.float32),
    )(W1r, obj)


def _sc_gather(table, idx_main, idx_tail):
    mesh = plsc.VectorSubcoreMesh(core_axis_name="c", subcore_axis_name="s")

    @functools.partial(
        pl.kernel,
        mesh=mesh,
        out_type=(
            jax.ShapeDtypeStruct((T, H), jnp.float32),
            jax.ShapeDtypeStruct((T, H), jnp.float32),
        ),
        scratch_types=[
            pltpu.VMEM((GCH, C), jnp.int32),
            pltpu.VMEM((GCH, C), jnp.int32),
            pltpu.VMEM((2, GTL), jnp.int32),
            pltpu.VMEM((C, H), jnp.float32),
            pltpu.VMEM((C, H), jnp.float32),
            pltpu.SemaphoreType.DMA,
            pltpu.SemaphoreType.DMA,
        ],
    )
    def k(table_hbm, idxm_hbm, idxt_hbm, outA_hbm, outC_hbm,
          idx_va, idx_vc, idx_vt, rows_a, rows_c, sem_a, sem_c):
        wid = lax.axis_index("s") * NC + lax.axis_index("c")
        base = wid * GPW
        pltpu.sync_copy(idxm_hbm.at[0, wid], idx_va)
        pltpu.sync_copy(idxm_hbm.at[1, wid], idx_vc)
        pltpu.sync_copy(idxt_hbm.at[0, wid], idx_vt.at[0])
        pltpu.sync_copy(idxt_hbm.at[1, wid], idx_vt.at[1])

        def body(j, carry):
            cp_a = pltpu.async_copy(table_hbm.at[idx_va.at[j]], rows_a, sem_a)
            cp_c = pltpu.async_copy(table_hbm.at[idx_vc.at[j]], rows_c, sem_c)
            cp_a.wait()
            pltpu.sync_copy(rows_a, outA_hbm.at[pl.ds(base + j * C, C)])
            cp_c.wait()
            pltpu.sync_copy(rows_c, outC_hbm.at[pl.ds(base + j * C, C)])
            return carry

        lax.fori_loop(0, GCH, body, 0)

        cp_a = pltpu.async_copy(table_hbm.at[idx_vt.at[0]],
                                rows_a.at[pl.ds(0, GTL)], sem_a)
        cp_c = pltpu.async_copy(table_hbm.at[idx_vt.at[1]],
                                rows_c.at[pl.ds(0, GTL)], sem_c)
        cp_a.wait()
        pltpu.sync_copy(rows_a.at[pl.ds(0, GTL)],
                        outA_hbm.at[pl.ds(base + GCH * C, GTL)])
        cp_c.wait()
        pltpu.sync_copy(rows_c.at[pl.ds(0, GTL)],
                        outC_hbm.at[pl.ds(base + GCH * C, GTL)])

    return k(table, idx_main, idx_tail)


def _edge_mlp(gA, gC, pred, W1b, b1r, W2, b2r):
    tile = 640
    n = T // tile

    def body(ga, gc, pr, w1, b1_, w2, b2_, os_, op_, oo_):
        h = ga[...] + gc[...] + b1_[...]
        h = h + jnp.dot(pr[...].astype(jnp.bfloat16), w1[...],
                        preferred_element_type=jnp.float32)
        h = jnp.maximum(h, 0.0)
        nt = jnp.dot(h.astype(jnp.bfloat16), w2[...],
                     preferred_element_type=jnp.float32) + b2_[...]
        nt = jnp.maximum(nt, 0.0)
        os_[...] = nt[:, :H]
        op_[...] = nt[:, H:H + D]
        oo_[...] = nt[:, H + D:]

    edge_spec = pl.BlockSpec((tile, D), lambda j: (j, 0))
    return pl.pallas_call(
        body,
        grid=(n,),
        in_specs=[
            edge_spec, edge_spec, edge_spec,
            pl.BlockSpec((D, H), lambda j: (0, 0)),
            pl.BlockSpec((1, H), lambda j: (0, 0)),
            pl.BlockSpec((H, 2 * H + D), lambda j: (0, 0)),
            pl.BlockSpec((1, 2 * H + D), lambda j: (0, 0)),
        ],
        out_specs=[edge_spec, edge_spec, edge_spec],
        out_shape=[
            jax.ShapeDtypeStruct((T, H), jnp.float32),
            jax.ShapeDtypeStruct((T, D), jnp.float32),
            jax.ShapeDtypeStruct((T, H), jnp.float32),
        ],
        compiler_params=pltpu.CompilerParams(
            dimension_semantics=("arbitrary",)),
    )(gA, gC, pred, W1b, b1r, W2, b2r)


def _sc_scatter_add(new_s, new_o, idx_main, idx_tail):
    mesh = plsc.VectorSubcoreMesh(core_axis_name="c", subcore_axis_name="s")

    @functools.partial(
        pl.kernel,
        mesh=mesh,
        out_type=jax.ShapeDtypeStruct((NC, O, H), jnp.float32),
        scratch_types=[
            pltpu.VMEM((2, SCH, C), jnp.int32),
            pltpu.VMEM((2, STL), jnp.int32),
            pltpu.VMEM((2, C, PW), jnp.float32),
            pltpu.VMEM_SHARED((O, PW), jnp.float32),
            pltpu.SemaphoreType.DMA,
            pltpu.SemaphoreType.DMA,
        ],
    )
    def k(s_hbm, o_hbm, idxm_hbm, idxt_hbm, out_hbm,
          idx_v, idx_vt, rows_v, acc, semA, semB):
        c = lax.axis_index("c")
        s = lax.axis_index("s")
        base = c * TPC + s * SPT

        pltpu.sync_copy(idxm_hbm.at[0, c, s], idx_v.at[0])
        pltpu.sync_copy(idxm_hbm.at[1, c, s], idx_v.at[1])
        pltpu.sync_copy(idxt_hbm.at[0, c, s], idx_vt.at[0])
        pltpu.sync_copy(idxt_hbm.at[1, c, s], idx_vt.at[1])

        for p in range(NP):
            col = pl.ds(p * PW, PW)

            def zrow(r, carry):
                for kk in range(PW // 16):
                    rows_v[0, r, pl.ds(kk * 16, 16)] = jnp.zeros(
                        (16,), jnp.float32)
                return carry

            lax.fori_loop(0, C, zrow, 0)
            z0 = 0
            for zr in (C, C, C, C, CPT - 4 * C):
                pltpu.sync_copy(rows_v.at[0, pl.ds(0, zr)],
                                acc.at[pl.ds(s * CPT + z0, zr)])
                z0 += zr

            @pl.when(s == NS - 1)
            def _():
                pltpu.sync_copy(rows_v.at[0, pl.ds(0, O - NS * CPT)],
                                acc.at[pl.ds(NS * CPT, O - NS * CPT)])

            plsc.subcore_barrier()

            for si, src_hbm in ((0, s_hbm), (1, o_hbm)):
                def cds(j):
                    return (pl.ds(base + j * C, C), col)

                pltpu.async_copy(src_hbm.at[cds(0)], rows_v.at[0], semA)

                def body2(kk, carry):
                    j0 = 2 * kk
                    pltpu.async_copy(src_hbm.at[cds(j0 + 1)],
                                     rows_v.at[1], semB)
                    pltpu.make_async_copy(src_hbm.at[cds(j0)],
                                          rows_v.at[0], semA).wait()
                    pltpu.sync_copy(rows_v.at[0],
                                    acc.at[idx_v.at[si, j0]], add=True)
                    pltpu.async_copy(src_hbm.at[cds(j0 + 2)],
                                     rows_v.at[0], semA)
                    pltpu.make_async_copy(src_hbm.at[cds(j0 + 1)],
                                          rows_v.at[1], semB).wait()
                    pltpu.sync_copy(rows_v.at[1],
                                    acc.at[idx_v.at[si, j0 + 1]], add=True)
                    return carry

                lax.fori_loop(0, SCH // 2, body2, 0)
                pltpu.make_async_copy(src_hbm.at[cds(SCH - 1)],
                                      rows_v.at[0], semA).wait()
                pltpu.sync_copy(rows_v.at[0],
                                acc.at[idx_v.at[si, SCH - 1]], add=True)
                pltpu.sync_copy(
                    src_hbm.at[pl.ds(base + SCH * C, STL), col],
                    rows_v.at[0, pl.ds(0, STL)])
                pltpu.sync_copy(rows_v.at[0, pl.ds(0, STL)],
                                acc.at[idx_vt.at[si]], add=True)

            plsc.subcore_barrier()

            pltpu.sync_copy(acc.at[pl.ds(s * CPT, CPT)],
                            out_hbm.at[c, pl.ds(s * CPT, CPT), col])

            @pl.when(s == NS - 1)
            def _():
                pltpu.sync_copy(
                    acc.at[pl.ds(NS * CPT, O - NS * CPT)],
                    out_hbm.at[c, pl.ds(NS * CPT, O - NS * CPT), col])

            plsc.subcore_barrier()

    return k(new_s, new_o, idx_main, idx_tail)


def _sumsq(partial):
    n = 25
    b = O // n

    def body(x_ref, o_ref, acc_ref):
        @pl.when(pl.program_id(0) == 0)
        def _():
            acc_ref[0] = 0.0

        x = x_ref[0] + x_ref[1]
        acc_ref[0] += jnp.sum(x * x)

        @pl.when(pl.program_id(0) == n - 1)
        def _():
            o_ref[...] = jnp.broadcast_to(acc_ref[0], (1, 1))

    return pl.pallas_call(
        body,
        grid=(n,),
        in_specs=[pl.BlockSpec((NC, b, H), lambda j: (0, j, 0))],
        out_specs=pl.BlockSpec((1, 1), lambda j: (0, 0)),
        out_shape=jax.ShapeDtypeStruct((1, 1), jnp.float32),
        scratch_shapes=[pltpu.SMEM((1,), jnp.float32)],
        compiler_params=pltpu.CompilerParams(
            dimension_semantics=("arbitrary",)),
    )(partial)


def _gconv2(partial, ss, W3, b3r, W4, b4r):
    n = 25
    b = O // n

    def body(ss_ref, x_ref, w3, b3_, w4, b4_, o_ref):
        inv = lax.rsqrt(ss_ref[0, 0])
        x = (x_ref[0] + x_ref[1]) * inv
        h = jnp.dot(x, w3[...],
                    preferred_element_type=jnp.float32) + b3_[...]
        h = jnp.maximum(h, 0.0)
        o = jnp.dot(h, w4[...], preferred_element_type=jnp.float32) + b4_[...]
        o_ref[...] = jnp.maximum(o, 0.0)

    return pl.pallas_call(
        body,
        grid=(n,),
        in_specs=[
            pl.BlockSpec((1, 1), lambda j: (0, 0)),
            pl.BlockSpec((NC, b, H), lambda j: (0, j, 0)),
            pl.BlockSpec((H, H), lambda j: (0, 0)),
            pl.BlockSpec((1, H), lambda j: (0, 0)),
            pl.BlockSpec((H, D), lambda j: (0, 0)),
            pl.BlockSpec((1, D), lambda j: (0, 0)),
        ],
        out_specs=pl.BlockSpec((b, D), lambda j: (j, 0)),
        out_shape=jax.ShapeDtypeStruct((O, D), jnp.float32),
    )(ss, partial, W3, b3r, W4, b4r)


def kernel(obj_vecs, pred_vecs, edges, W1, b1, W2, b2, W3, b3, W4, b4):
    obj = obj_vecs[0]
    pred = pred_vecs[0]
    s_idx = edges[0, :, 0].astype(jnp.int32)
    o_idx = edges[0, :, 1].astype(jnp.int32)

    W1r = W1.reshape(3, D, H)

    table = _proj_table(obj, W1r)

    idx_g = jnp.stack([s_idx, o_idx + O]).reshape(2, NW, GPW)
    idx_gm = idx_g[:, :, :GCH * C].reshape(2, NW, GCH, C)
    idx_gt = idx_g[:, :, GCH * C:]
    gA, gC = _sc_gather(table, idx_gm, idx_gt)

    new_s, new_pred, new_o = _edge_mlp(
        gA, gC, pred, W1r[1].astype(jnp.bfloat16), b1.reshape(1, H),
        W2.astype(jnp.bfloat16), b2.reshape(1, 2 * H + D))

    idx_sc = jnp.stack([s_idx, o_idx]).reshape(2, NC, NS, SPT)
    idx_sm = idx_sc[..., :SCH * C].reshape(2, NC, NS, SCH, C)
    idx_st = idx_sc[..., SCH * C:]
    partial = _sc_scatter_add(new_s, new_o, idx_sm, idx_st)

    ss = _sumsq(partial)
    new_obj = _gconv2(partial, ss, W3, b3.reshape(1, H), W4, b4.reshape(1, D))

    return new_obj[None], new_pred[None]
